# baseline (speedup 1.0000x reference)
import jax
import jax.numpy as jnp
from jax import lax
from jax.experimental import pallas as pl
from jax.experimental.pallas import tpu as pltpu
from jax.experimental.pallas import tpu_sc as plsc

N = 10000
E = 320000
D = 128
MAX_NORM = 1.0 - 1e-5
MIN_NORM = 1e-15

NC = 2
NS = 16
NW = NC * NS
EPW = E // NW
K = 80
NCHUNK = 126
EPAD = NCHUNK * K

ROW_BLK = 1000


def _norm_cols(x):
    return jnp.sqrt(jnp.sum(x * x, axis=-1, keepdims=True))


def _head_body(x_ref, w_ref, o_ref):
    x = x_ref[...]
    n = jnp.clip(_norm_cols(x), MIN_NORM, None)
    nc = jnp.clip(n, None, MAX_NORM)
    atanh = 0.5 * jnp.log((1.0 + nc) / (1.0 - nc))
    t = atanh * x / n
    m = jnp.dot(t, w_ref[...], preferred_element_type=jnp.float32)
    m4 = m.reshape(ROW_BLK, D // 32, 2, 16)
    swz = jnp.stack([m4[:, :, 0, :], m4[:, :, 1, :]], axis=-1)
    o_ref[...] = swz.reshape(ROW_BLK, D).astype(jnp.bfloat16)


def _head(x, w):
    return pl.pallas_call(
        _head_body,
        grid=(N // ROW_BLK,),
        in_specs=[
            pl.BlockSpec((ROW_BLK, D), lambda i: (i, 0)),
            pl.BlockSpec((D, D), lambda i: (0, 0)),
        ],
        out_specs=pl.BlockSpec((ROW_BLK, D), lambda i: (i, 0)),
        out_shape=jax.ShapeDtypeStruct((N, D), jnp.bfloat16),
    )(x, w)


def _agg_body(mapped_hbm, src_hbm, dst_hbm, adj_hbm, zeros_hbm, out_hbm,
              src_v, adj_v, dstb0, dstb1, rows0, rows1, out_f, acc_sh,
              g0, g1, d0, d1):
    rows = [rows0, rows1]
    dstb = [dstb0, dstb1]
    gsem = [g0, g1]
    dsem = [d0, d1]
    cid = lax.axis_index("c")
    sid = lax.axis_index("s")
    wid = cid * NS + sid
    rpz = (N // NS) // 8 * 8
    rem = N - NS * rpz

    pltpu.sync_copy(zeros_hbm.at[pl.ds(sid * rpz, rpz)],
                    acc_sh.at[pl.ds(sid * rpz, rpz)])

    @pl.when(sid == NS - 1)
    def _zero_rem():
        pltpu.sync_copy(zeros_hbm.at[pl.ds(NS * rpz, rem)],
                        acc_sh.at[pl.ds(NS * rpz, rem)])
    pltpu.sync_copy(src_hbm.at[wid], src_v)
    pltpu.sync_copy(adj_hbm.at[wid], adj_v)
    plsc.subcore_barrier()

    pltpu.async_copy(mapped_hbm.at[src_v.at[pl.ds(0, K)]], rows[0], gsem[0])
    pltpu.async_copy(dst_hbm.at[wid, 0], dstb[0], dsem[0])

    def outer_body(oc, carry):
        for b in range(2):
            c = oc * 2 + b
            b1 = 1 - b

            pltpu.make_async_copy(
                mapped_hbm.at[src_v.at[pl.ds(c * K, K)]],
                rows[b], gsem[b]).wait()

            @pl.when(c + 1 < NCHUNK)
            def _issue_next():
                pltpu.async_copy(
                    mapped_hbm.at[src_v.at[pl.ds((c + 1) * K, K)]],
                    rows[b1], gsem[b1])
                pltpu.async_copy(dst_hbm.at[wid, c + 1], dstb[b1], dsem[b1])

            def group_body(g, carry2):
                av = adj_v[pl.ds(c * K + g * 16, 16)]
                for r in range(16):
                    a = av[r]
                    e = g * 16 + r
                    for j in range(D // 32):
                        vi = rows[b][e, pl.ds(j * 16, 16)]
                        vb = plsc.bitcast(vi, jnp.bfloat16)
                        lo, hi = plsc.unpack(
                            vb, format=plsc.PackFormat.INTERLEAVED)
                        out_f[e, pl.ds(j * 32, 16)] = lo * a
                        out_f[e, pl.ds(j * 32 + 16, 16)] = hi * a
                return carry2

            lax.fori_loop(0, K // 16, group_body, 0)

            pltpu.make_async_copy(dst_hbm.at[wid, c], dstb[b], dsem[b]).wait()
            pltpu.sync_copy(out_f, acc_sh.at[dstb[b]], add=True)
        return carry

    lax.fori_loop(0, NCHUNK // 2, outer_body, 0)
    plsc.subcore_barrier()
    pltpu.sync_copy(acc_sh.at[pl.ds(sid * rpz, rpz)],
                    out_hbm.at[cid, pl.ds(sid * rpz, rpz)])

    @pl.when(sid == NS - 1)
    def _write_rem():
        pltpu.sync_copy(acc_sh.at[pl.ds(NS * rpz, rem)],
                        out_hbm.at[cid, pl.ds(NS * rpz, rem)])


def _agg(mapped, src, dst, adj, zeros):
    mesh = plsc.VectorSubcoreMesh(core_axis_name="c", subcore_axis_name="s")
    f = pl.kernel(
        _agg_body,
        out_type=jax.ShapeDtypeStruct((NC, N, D), jnp.float32),
        mesh=mesh,
        compiler_params=pltpu.CompilerParams(use_tc_tiling_on_sc=False,
                                             needs_layout_passes=False),
        scratch_types=[
            pltpu.VMEM((EPAD,), jnp.int32),
            pltpu.VMEM((EPAD,), jnp.float32),
            pltpu.VMEM((K,), jnp.int32),
            pltpu.VMEM((K,), jnp.int32),
            pltpu.VMEM((K, D // 2), jnp.int32),
            pltpu.VMEM((K, D // 2), jnp.int32),
            pltpu.VMEM((K, D), jnp.float32),
            pltpu.VMEM_SHARED((N, D), jnp.float32),
            pltpu.SemaphoreType.DMA,
            pltpu.SemaphoreType.DMA,
            pltpu.SemaphoreType.DMA,
            pltpu.SemaphoreType.DMA,
        ],
    )
    return f(mapped, src, dst, adj, zeros)


def _tail_body(agg_ref, bias_ref, o_ref):
    agg = agg_ref[0] + agg_ref[1]
    n = jnp.clip(_norm_cols(agg), MIN_NORM, None)
    e = jnp.tanh(n) * agg / n
    ne = jnp.clip(_norm_cols(e), MIN_NORM, None)
    x = e * jnp.where(ne > MAX_NORM, MAX_NORM / ne, 1.0)
    bv = bias_ref[...]
    nb = jnp.clip(_norm_cols(bv), MIN_NORM, None)
    b = jnp.tanh(nb) * bv / nb
    nb2 = jnp.clip(_norm_cols(b), MIN_NORM, None)
    b = b * jnp.where(nb2 > MAX_NORM, MAX_NORM / nb2, 1.0)
    xy = jnp.sum(x * b, axis=-1, keepdims=True)
    x2 = jnp.sum(x * x, axis=-1, keepdims=True)
    y2 = jnp.sum(b * b, axis=-1, keepdims=True)
    num = (1.0 + 2.0 * xy + y2) * x + (1.0 - x2) * b
    den = jnp.clip(1.0 + 2.0 * xy + x2 * y2, MIN_NORM, None)
    out = num / den
    no = jnp.clip(_norm_cols(out), MIN_NORM, None)
    o_ref[...] = out * jnp.where(no > MAX_NORM, MAX_NORM / no, 1.0)


def _tail(agg2, bias_vec):
    return pl.pallas_call(
        _tail_body,
        grid=(N // ROW_BLK,),
        in_specs=[
            pl.BlockSpec((NC, ROW_BLK, D), lambda i: (0, i, 0)),
            pl.BlockSpec((1, D), lambda i: (0, 0)),
        ],
        out_specs=pl.BlockSpec((ROW_BLK, D), lambda i: (i, 0)),
        out_shape=jax.ShapeDtypeStruct((N, D), jnp.float32),
    )(agg2, bias_vec)


def kernel(ents_embed_input, W_ent, bias_vec, edge_index, adj_values):
    mapped_bf = _head(ents_embed_input, W_ent)
    mapped = jax.lax.bitcast_convert_type(
        mapped_bf.reshape(N, D // 2, 2), jnp.int32)

    def _shape_edges(x, flat):
        xp = jnp.pad(x.reshape(NW, EPW), ((0, 0), (0, EPAD - EPW)))
        return xp if flat else xp.reshape(NW, NCHUNK, K)

    src = _shape_edges(edge_index[1], True)
    dst = _shape_edges(edge_index[0], False)
    adj = _shape_edges(adj_values, True)
    zeros = jnp.zeros((N, D), jnp.float32)
    agg2 = _agg(mapped, src, dst, adj, zeros)
    return _tail(agg2, bias_vec)


# speedup vs baseline: 2.3172x; 2.3172x over previous
import jax
import jax.numpy as jnp
from jax import lax
from jax.experimental import pallas as pl
from jax.experimental.pallas import tpu as pltpu
from jax.experimental.pallas import tpu_sc as plsc

N = 10000
E = 320000
D = 128
MAX_NORM = 1.0 - 1e-5
MIN_NORM = 1e-15

NC = 2
NS = 16
NW = NC * NS
EPW = E // NW
K = 112
NCHUNK = 90
EPAD = NCHUNK * K

ROW_BLK = 1000


def _norm_cols(x):
    return jnp.sqrt(jnp.sum(x * x, axis=-1, keepdims=True))


def _head_body(x_ref, w_ref, o_ref):
    x = x_ref[...]
    n = jnp.clip(_norm_cols(x), MIN_NORM, None)
    nc = jnp.clip(n, None, MAX_NORM)
    atanh = 0.5 * jnp.log((1.0 + nc) / (1.0 - nc))
    t = atanh * x / n
    o_ref[...] = jnp.dot(t, w_ref[...], preferred_element_type=jnp.float32)


def _head(x, w):
    return pl.pallas_call(
        _head_body,
        grid=(N // ROW_BLK,),
        in_specs=[
            pl.BlockSpec((ROW_BLK, D), lambda i: (i, 0)),
            pl.BlockSpec((D, D), lambda i: (0, 0)),
        ],
        out_specs=pl.BlockSpec((ROW_BLK, D), lambda i: (i, 0)),
        out_shape=jax.ShapeDtypeStruct((N, D), jnp.float32),
    )(x, w)


def _agg_body(mapped_hbm, src_hbm, dst_hbm, adj_hbm, zeros_hbm, out_hbm,
              src_v, adj_v, dstb0, dstb1, rows0, rows1, acc_sh,
              g0, g1, d0, d1):
    rows = [rows0, rows1]
    dstb = [dstb0, dstb1]
    gsem = [g0, g1]
    dsem = [d0, d1]
    cid = lax.axis_index("c")
    sid = lax.axis_index("s")
    wid = cid * NS + sid
    rpz = (N // NS) // 8 * 8
    rem = N - NS * rpz

    pltpu.sync_copy(zeros_hbm.at[pl.ds(sid * rpz, rpz)],
                    acc_sh.at[pl.ds(sid * rpz, rpz)])

    @pl.when(sid == NS - 1)
    def _zero_rem():
        pltpu.sync_copy(zeros_hbm.at[pl.ds(NS * rpz, rem)],
                        acc_sh.at[pl.ds(NS * rpz, rem)])
    pltpu.sync_copy(src_hbm.at[wid], src_v)
    pltpu.sync_copy(adj_hbm.at[wid], adj_v)
    plsc.subcore_barrier()

    pltpu.async_copy(mapped_hbm.at[src_v.at[pl.ds(0, K)]], rows[0], gsem[0])
    pltpu.async_copy(dst_hbm.at[wid, 0], dstb[0], dsem[0])

    def outer_body(oc, carry):
        for b in range(2):
            c = oc * 2 + b
            b1 = 1 - b

            pltpu.make_async_copy(
                mapped_hbm.at[src_v.at[pl.ds(c * K, K)]],
                rows[b], gsem[b]).wait()

            @pl.when(c + 1 < NCHUNK)
            def _issue_next():
                pltpu.async_copy(
                    mapped_hbm.at[src_v.at[pl.ds((c + 1) * K, K)]],
                    rows[b1], gsem[b1])
                pltpu.async_copy(dst_hbm.at[wid, c + 1], dstb[b1], dsem[b1])

            def group_body(g, carry2):
                av = adj_v[pl.ds(c * K + g * 16, 16)]
                for r in range(16):
                    a = av[r]
                    e = g * 16 + r
                    for j in range(D // 16):
                        sl = pl.ds(j * 16, 16)
                        rows[b][e, sl] = rows[b][e, sl] * a
                return carry2

            lax.fori_loop(0, K // 16, group_body, 0)

            pltpu.make_async_copy(dst_hbm.at[wid, c], dstb[b], dsem[b]).wait()
            pltpu.sync_copy(rows[b], acc_sh.at[dstb[b]], add=True)
        return carry

    lax.fori_loop(0, NCHUNK // 2, outer_body, 0)
    plsc.subcore_barrier()
    pltpu.sync_copy(acc_sh.at[pl.ds(sid * rpz, rpz)],
                    out_hbm.at[cid, pl.ds(sid * rpz, rpz)])

    @pl.when(sid == NS - 1)
    def _write_rem():
        pltpu.sync_copy(acc_sh.at[pl.ds(NS * rpz, rem)],
                        out_hbm.at[cid, pl.ds(NS * rpz, rem)])


def _agg(mapped, src, dst, adj, zeros):
    mesh = plsc.VectorSubcoreMesh(core_axis_name="c", subcore_axis_name="s")
    f = pl.kernel(
        _agg_body,
        out_type=jax.ShapeDtypeStruct((NC, N, D), jnp.float32),
        mesh=mesh,
        scratch_types=[
            pltpu.VMEM((EPAD,), jnp.int32),
            pltpu.VMEM((EPAD,), jnp.float32),
            pltpu.VMEM((K,), jnp.int32),
            pltpu.VMEM((K,), jnp.int32),
            pltpu.VMEM((K, D), jnp.float32),
            pltpu.VMEM((K, D), jnp.float32),
            pltpu.VMEM_SHARED((N, D), jnp.float32),
            pltpu.SemaphoreType.DMA,
            pltpu.SemaphoreType.DMA,
            pltpu.SemaphoreType.DMA,
            pltpu.SemaphoreType.DMA,
        ],
    )
    return f(mapped, src, dst, adj, zeros)


def _tail_body(agg_ref, bias_ref, o_ref):
    agg = agg_ref[0] + agg_ref[1]
    n = jnp.clip(_norm_cols(agg), MIN_NORM, None)
    e = jnp.tanh(n) * agg / n
    ne = jnp.clip(_norm_cols(e), MIN_NORM, None)
    x = e * jnp.where(ne > MAX_NORM, MAX_NORM / ne, 1.0)
    bv = bias_ref[...]
    nb = jnp.clip(_norm_cols(bv), MIN_NORM, None)
    b = jnp.tanh(nb) * bv / nb
    nb2 = jnp.clip(_norm_cols(b), MIN_NORM, None)
    b = b * jnp.where(nb2 > MAX_NORM, MAX_NORM / nb2, 1.0)
    xy = jnp.sum(x * b, axis=-1, keepdims=True)
    x2 = jnp.sum(x * x, axis=-1, keepdims=True)
    y2 = jnp.sum(b * b, axis=-1, keepdims=True)
    num = (1.0 + 2.0 * xy + y2) * x + (1.0 - x2) * b
    den = jnp.clip(1.0 + 2.0 * xy + x2 * y2, MIN_NORM, None)
    out = num / den
    no = jnp.clip(_norm_cols(out), MIN_NORM, None)
    o_ref[...] = out * jnp.where(no > MAX_NORM, MAX_NORM / no, 1.0)


def _tail(agg2, bias_vec):
    return pl.pallas_call(
        _tail_body,
        grid=(N // ROW_BLK,),
        in_specs=[
            pl.BlockSpec((NC, ROW_BLK, D), lambda i: (0, i, 0)),
            pl.BlockSpec((1, D), lambda i: (0, 0)),
        ],
        out_specs=pl.BlockSpec((ROW_BLK, D), lambda i: (i, 0)),
        out_shape=jax.ShapeDtypeStruct((N, D), jnp.float32),
    )(agg2, bias_vec)


def kernel(ents_embed_input, W_ent, bias_vec, edge_index, adj_values):
    mapped = _head(ents_embed_input, W_ent)

    def _shape_edges(x, flat):
        xp = jnp.pad(x.reshape(NW, EPW), ((0, 0), (0, EPAD - EPW)))
        return xp if flat else xp.reshape(NW, NCHUNK, K)

    src = _shape_edges(edge_index[1], True)
    dst = _shape_edges(edge_index[0], False)
    adj = _shape_edges(adj_values, True)
    zeros = jnp.zeros((N, D), jnp.float32)
    agg2 = _agg(mapped, src, dst, adj, zeros)
    return _tail(agg2, bias_vec)
